# baseline (speedup 1.0000x reference)
import jax
import jax.numpy as jnp
from jax.experimental import pallas as pl

N_NODES = 10000
K = 32
N_SEEDS = 2500


def _fps(pos, n_samples):
    pos = jax.lax.stop_gradient(pos)
    N = pos.shape[0]

    def body(i, state):
        dists, idxs, cur = state
        idxs = idxs.at[i].set(cur)
        d = jnp.sum((pos - pos[cur]) ** 2, axis=1)
        dists = jnp.minimum(dists, d)
        cur = jnp.argmax(dists).astype(jnp.int32)
        return (dists, idxs, cur)

    dists0 = jnp.full((N,), jnp.inf, dtype=jnp.float32)
    idxs0 = jnp.zeros((n_samples,), dtype=jnp.int32)
    _, idxs, _ = jax.lax.fori_loop(0, n_samples, body, (dists0, idxs0, jnp.int32(0)))
    return idxs


def _knn(pos, seeds, k):
    d2 = (jnp.sum(seeds ** 2, axis=1, keepdims=True)
          + jnp.sum(pos ** 2, axis=1)[None, :]
          - 2.0 * seeds @ pos.T)
    _, idx = jax.lax.top_k(-d2, k)
    return idx


def _bn(h, g, b):
    mean = jnp.mean(h, axis=0)
    var = jnp.mean((h - mean) ** 2, axis=0)
    return (h - mean) / jnp.sqrt(var + 1e-5) * g + b


def _copy_body(i_ref, o_ref):
    o_ref[...] = i_ref[...]


def kernel(x, pos, batch, W1a, b1a, g1, be1, W1b, b1b, W2a, b2a, g2, be2, W2b, b2b):
    seed_idx = _fps(pos, N_SEEDS)
    seeds = pos[seed_idx]
    nbr = _knn(pos, seeds, K)
    to_idx = nbr.reshape(-1)
    from_idx = jnp.repeat(jnp.arange(N_SEEDS, dtype=jnp.int32), K)
    pos_j = pos[to_idx]
    pos_i = seeds[from_idx]
    msg = pos_j - pos_i
    h = _bn(msg @ W1a + b1a, g1, be1)
    h = jax.nn.relu(h)
    h = h @ W1b + b1b
    h = h.reshape(-1, K, 256)
    hmax = jnp.max(h, axis=1, keepdims=True)
    h = jnp.concatenate([jnp.broadcast_to(hmax, h.shape), h], axis=2).reshape(-1, 512)
    h2 = _bn(h @ W2a + b2a, g2, be2)
    h2 = jax.nn.relu(h2)
    h2 = h2 @ W2b + b2b
    out = jax.ops.segment_max(h2, from_idx, num_segments=N_SEEDS)
    return pl.pallas_call(
        _copy_body,
        out_shape=jax.ShapeDtypeStruct((N_SEEDS, 128), jnp.float32),
    )(out)


# speedup vs baseline: 3.4579x; 3.4579x over previous
import functools

import jax
import jax.numpy as jnp
from jax.experimental import pallas as pl

N_NODES = 10000
K = 32
N_SEEDS = 2500
NPAD = 10240
FPS_R = 8
FPS_C = 1280
SEED_R = 2
SEED_C = 1280


def _fps_body(n_samples, x_ref, y_ref, z_ref, si_ref, sx_ref, sy_ref, sz_ref):
    lin = (jax.lax.broadcasted_iota(jnp.int32, (FPS_R, FPS_C), 0) * FPS_C
           + jax.lax.broadcasted_iota(jnp.int32, (FPS_R, FPS_C), 1))
    lin2 = (jax.lax.broadcasted_iota(jnp.int32, (SEED_R, SEED_C), 0) * SEED_C
            + jax.lax.broadcasted_iota(jnp.int32, (SEED_R, SEED_C), 1))
    valid = lin < N_NODES
    dists0 = jnp.where(valid, jnp.inf, -jnp.inf).astype(jnp.float32)
    zi = jnp.zeros((SEED_R, SEED_C), jnp.int32)
    zf = jnp.zeros((SEED_R, SEED_C), jnp.float32)

    def body(i, carry):
        dists, cur, si, sx, sy, sz = carry
        xp = x_ref[...]
        yp = y_ref[...]
        zp = z_ref[...]
        m = lin == cur
        mf = m.astype(jnp.float32)
        cx = jnp.sum(mf * xp)
        cy = jnp.sum(mf * yp)
        cz = jnp.sum(mf * zp)
        rec = lin2 == i
        si = jnp.where(rec, cur, si)
        sx = jnp.where(rec, cx, sx)
        sy = jnp.where(rec, cy, sy)
        sz = jnp.where(rec, cz, sz)
        dx = xp - cx
        dy = yp - cy
        dz = zp - cz
        d = (dx * dx + dy * dy) + dz * dz
        dists = jnp.minimum(dists, d)
        mx = jnp.max(dists)
        cur = jnp.min(jnp.where(dists == mx, lin, jnp.int32(2 ** 30)))
        return dists, cur, si, sx, sy, sz

    _, _, si, sx, sy, sz = jax.lax.fori_loop(
        0, n_samples, body, (dists0, jnp.int32(0), zi, zf, zf, zf))
    si_ref[...] = si
    sx_ref[...] = sx
    sy_ref[...] = sy
    sz_ref[...] = sz


def _fps_pallas(pos, n_samples):
    pad = jnp.zeros((NPAD - N_NODES,), jnp.float32)
    xp = jnp.concatenate([pos[:, 0], pad]).reshape(FPS_R, FPS_C)
    yp = jnp.concatenate([pos[:, 1], pad]).reshape(FPS_R, FPS_C)
    zp = jnp.concatenate([pos[:, 2], pad]).reshape(FPS_R, FPS_C)
    si, sx, sy, sz = pl.pallas_call(
        functools.partial(_fps_body, n_samples),
        out_shape=[
            jax.ShapeDtypeStruct((SEED_R, SEED_C), jnp.int32),
            jax.ShapeDtypeStruct((SEED_R, SEED_C), jnp.float32),
            jax.ShapeDtypeStruct((SEED_R, SEED_C), jnp.float32),
            jax.ShapeDtypeStruct((SEED_R, SEED_C), jnp.float32),
        ],
    )(xp, yp, zp)
    seed_idx = si.reshape(-1)[:n_samples]
    seeds = jnp.stack([sx.reshape(-1)[:n_samples],
                       sy.reshape(-1)[:n_samples],
                       sz.reshape(-1)[:n_samples]], axis=1)
    return seed_idx, seeds


def _knn(pos, seeds, k):
    d2 = (jnp.sum(seeds ** 2, axis=1, keepdims=True)
          + jnp.sum(pos ** 2, axis=1)[None, :]
          - 2.0 * seeds @ pos.T)
    _, idx = jax.lax.top_k(-d2, k)
    return idx


def _bn(h, g, b):
    mean = jnp.mean(h, axis=0)
    var = jnp.mean((h - mean) ** 2, axis=0)
    return (h - mean) / jnp.sqrt(var + 1e-5) * g + b


def kernel(x, pos, batch, W1a, b1a, g1, be1, W1b, b1b, W2a, b2a, g2, be2, W2b, b2b):
    seed_idx, seeds = _fps_pallas(pos, N_SEEDS)
    nbr = _knn(pos, seeds, K)
    to_idx = nbr.reshape(-1)
    from_idx = jnp.repeat(jnp.arange(N_SEEDS, dtype=jnp.int32), K)
    pos_j = pos[to_idx]
    pos_i = seeds[from_idx]
    msg = pos_j - pos_i
    h = _bn(msg @ W1a + b1a, g1, be1)
    h = jax.nn.relu(h)
    h = h @ W1b + b1b
    h = h.reshape(-1, K, 256)
    hmax = jnp.max(h, axis=1, keepdims=True)
    h = jnp.concatenate([jnp.broadcast_to(hmax, h.shape), h], axis=2).reshape(-1, 512)
    h2 = _bn(h @ W2a + b2a, g2, be2)
    h2 = jax.nn.relu(h2)
    h2 = h2 @ W2b + b2b
    out = jax.ops.segment_max(h2, from_idx, num_segments=N_SEEDS)
    return out


# speedup vs baseline: 3.6009x; 1.0414x over previous
import functools

import jax
import jax.numpy as jnp
from jax import lax
from jax.experimental import pallas as pl
from jax.experimental.pallas import tpu as pltpu
from jax.experimental.pallas import tpu_sc as plsc

N_NODES = 10000
K = 32
N_SEEDS = 2500
NPAD = 10240
FPS_R = 8
FPS_C = 1280
SEED_R = 2
SEED_C = 1280
NSPAD = 2560
NW = 32
SPW = NSPAD // NW
NV = NPAD // 16
CAND_CAP = 2048


def _fps_body(n_samples, x_ref, y_ref, z_ref, si_ref, sx_ref, sy_ref, sz_ref):
    lin = (lax.broadcasted_iota(jnp.int32, (FPS_R, FPS_C), 0) * FPS_C
           + lax.broadcasted_iota(jnp.int32, (FPS_R, FPS_C), 1))
    lin2 = (lax.broadcasted_iota(jnp.int32, (SEED_R, SEED_C), 0) * SEED_C
            + lax.broadcasted_iota(jnp.int32, (SEED_R, SEED_C), 1))
    valid = lin < N_NODES
    dists0 = jnp.where(valid, jnp.inf, -jnp.inf).astype(jnp.float32)
    zi = jnp.zeros((SEED_R, SEED_C), jnp.int32)
    zf = jnp.zeros((SEED_R, SEED_C), jnp.float32)

    def body(i, carry):
        dists, cur, si, sx, sy, sz = carry
        xp = x_ref[...]
        yp = y_ref[...]
        zp = z_ref[...]
        m = lin == cur
        mf = m.astype(jnp.float32)
        cx = jnp.sum(mf * xp)
        cy = jnp.sum(mf * yp)
        cz = jnp.sum(mf * zp)
        rec = lin2 == i
        si = jnp.where(rec, cur, si)
        sx = jnp.where(rec, cx, sx)
        sy = jnp.where(rec, cy, sy)
        sz = jnp.where(rec, cz, sz)
        dx = xp - cx
        dy = yp - cy
        dz = zp - cz
        d = (dx * dx + dy * dy) + dz * dz
        dists = jnp.minimum(dists, d)
        mx = jnp.max(dists)
        cur = jnp.min(jnp.where(dists == mx, lin, jnp.int32(2 ** 30)))
        return dists, cur, si, sx, sy, sz

    _, _, si, sx, sy, sz = lax.fori_loop(
        0, n_samples, body, (dists0, jnp.int32(0), zi, zf, zf, zf))
    si_ref[...] = si
    sx_ref[...] = sx
    sy_ref[...] = sy
    sz_ref[...] = sz


def _fps_pallas(pos, n_samples):
    pad = jnp.zeros((NPAD - N_NODES,), jnp.float32)
    xp = jnp.concatenate([pos[:, 0], pad]).reshape(FPS_R, FPS_C)
    yp = jnp.concatenate([pos[:, 1], pad]).reshape(FPS_R, FPS_C)
    zp = jnp.concatenate([pos[:, 2], pad]).reshape(FPS_R, FPS_C)
    si, sx, sy, sz = pl.pallas_call(
        functools.partial(_fps_body, n_samples),
        out_shape=[
            jax.ShapeDtypeStruct((SEED_R, SEED_C), jnp.int32),
            jax.ShapeDtypeStruct((SEED_R, SEED_C), jnp.float32),
            jax.ShapeDtypeStruct((SEED_R, SEED_C), jnp.float32),
            jax.ShapeDtypeStruct((SEED_R, SEED_C), jnp.float32),
        ],
    )(xp, yp, zp)
    return si, sx, sy, sz


def _d2_body(s_ref, p_ref, o_ref):
    j = pl.program_id(1)
    d = jnp.dot(s_ref[...], p_ref[...], preferred_element_type=jnp.float32)
    col = j * 2560 + lax.broadcasted_iota(jnp.int32, d.shape, 1)
    o_ref[...] = jnp.where(col < N_NODES, d, jnp.inf)


def _d2_pallas(S, P):
    return pl.pallas_call(
        _d2_body,
        grid=(5, 4),
        in_specs=[pl.BlockSpec((512, 8), lambda i, j: (i, 0)),
                  pl.BlockSpec((8, 2560), lambda i, j: (0, j))],
        out_specs=pl.BlockSpec((512, 2560), lambda i, j: (i, j)),
        out_shape=jax.ShapeDtypeStruct((NSPAD, NPAD), jnp.float32),
    )(S, P)


def _merge16(x, y):
    yr = lax.rev(y, (0,))
    lo = jnp.minimum(x, yr)
    hi = jnp.maximum(x, yr)
    return lax.sort(lo), lax.sort(hi)


def _sc_topk_gather_body(d2_hbm, pos_hbm, posj_hbm,
                         rb, candv, candi, nbrb, oidxb, gbuf, sem0, sem1, semg):
    cid = lax.axis_index("c")
    sid = lax.axis_index("s")
    wid = sid * 2 + cid
    s0 = wid * SPW
    iota = lax.iota(jnp.int32, 16)
    inf_v = jnp.full((16,), jnp.inf, jnp.float32)

    def fill_oidx(j, _):
        def inner(t, _):
            kvec = iota + 16 * (t % 2)
            r = 4 * j + t // 2
            oidxb[j, pl.ds(16 * t, 16)] = kvec * NSPAD + s0 + r
            return 0
        return lax.fori_loop(0, 8, inner, 0)
    lax.fori_loop(0, SPW // 4, fill_oidx, 0)

    def process_row(buf, r):
        def p1(j, carry):
            m0, m1 = carry
            m0 = jnp.minimum(m0, rb[buf, pl.ds(32 * j, 16)])
            m1 = jnp.minimum(m1, rb[buf, pl.ds(32 * j + 16, 16)])
            return m0, m1
        m0, m1 = lax.fori_loop(0, NV // 2, p1, (inf_v, inf_v), unroll=8)
        tau = jnp.max(jnp.maximum(m0, m1))

        def p2(j, off):
            v = rb[buf, pl.ds(16 * j, 16)]
            m = v <= tau
            offc = jnp.minimum(off, CAND_CAP - 16)
            plsc.store_compressed(candv.at[pl.ds(offc, 16)], v, mask=m)
            plsc.store_compressed(candi.at[pl.ds(offc, 16)], iota + 16 * j, mask=m)
            return offc + jnp.sum(m.astype(jnp.int32))
        off_n = lax.fori_loop(0, NV, p2, jnp.int32(0), unroll=4)
        candv[pl.ds(off_n, 16)] = inf_v
        nv = (off_n + 15) // 16

        a = lax.sort(candv[pl.ds(0, 16)])
        b = lax.sort(candv[pl.ds(16, 16)])
        a, b = _merge16(a, b)

        def sel(j, ab):
            aa, bb = ab
            c = lax.sort(candv[pl.ds(16 * j, 16)])
            l1, h1 = _merge16(bb, c)
            a2, m = _merge16(aa, l1)
            b2, _ = _merge16(m, h1)
            return a2, b2
        a, b = lax.fori_loop(2, nv, sel, (a, b))
        t32 = jnp.max(b)

        base = r * K

        def em1(j, off2):
            v = candv[pl.ds(16 * j, 16)]
            i = candi[pl.ds(16 * j, 16)]
            m = v < t32
            plsc.store_compressed(nbrb.at[pl.ds(base + off2, 16)], i, mask=m)
            return off2 + jnp.sum(m.astype(jnp.int32))
        off2 = lax.fori_loop(0, nv, em1, jnp.int32(0))

        def em2(j, offe):
            v = candv[pl.ds(16 * j, 16)]
            i = candi[pl.ds(16 * j, 16)]
            m = v == t32
            offc = jnp.minimum(offe, 48)
            plsc.store_compressed(candi.at[pl.ds(CAND_CAP - 64 + offc, 16)], i, mask=m)
            return offc + jnp.sum(m.astype(jnp.int32))
        lax.fori_loop(0, nv, em2, jnp.int32(0))

        need = K - off2
        e0 = candi[pl.ds(CAND_CAP - 64, 16)]
        plsc.store_compressed(nbrb.at[pl.ds(base + off2, 16)], e0, mask=iota < need)

        @pl.when(need > 16)
        def _():
            e1 = candi[pl.ds(CAND_CAP - 48, 16)]
            plsc.store_compressed(nbrb.at[pl.ds(base + off2 + 16, 16)], e1,
                                  mask=iota < need - 16)

    pltpu.make_async_copy(d2_hbm.at[s0], rb.at[0], sem0).start()

    def row_pair(r2, _):
        pltpu.make_async_copy(d2_hbm.at[s0 + 2 * r2 + 1], rb.at[1], sem1).start()
        pltpu.make_async_copy(d2_hbm.at[s0 + 2 * r2], rb.at[0], sem0).wait()
        process_row(0, 2 * r2)

        @pl.when(r2 < SPW // 2 - 1)
        def _():
            pltpu.make_async_copy(d2_hbm.at[s0 + 2 * r2 + 2], rb.at[0], sem0).start()
        pltpu.make_async_copy(d2_hbm.at[s0 + 2 * r2 + 1], rb.at[1], sem1).wait()
        process_row(1, 2 * r2 + 1)
        return 0

    lax.fori_loop(0, SPW // 2, row_pair, 0)

    def gs(j, _):
        pltpu.make_async_copy(pos_hbm.at[nbrb.at[pl.ds(128 * j, 128)]],
                              gbuf.at[pl.ds(128 * j, 128)], semg).wait()
        return 0
    lax.fori_loop(0, (SPW * K) // 128, gs, 0)

    def sc(j, _):
        pltpu.make_async_copy(gbuf.at[pl.ds(128 * j, 128)],
                              posj_hbm.at[oidxb.at[j]], semg).wait()
        return 0
    lax.fori_loop(0, (SPW * K) // 128, sc, 0)


def _sc_topk_gather(d2, pos16):
    mesh = plsc.VectorSubcoreMesh(core_axis_name="c", subcore_axis_name="s")
    f = pl.kernel(
        _sc_topk_gather_body,
        out_type=jax.ShapeDtypeStruct((K * NSPAD, 16), jnp.float32),
        mesh=mesh,
        compiler_params=pltpu.CompilerParams(needs_layout_passes=False),
        scratch_types=[
            pltpu.VMEM((2, NPAD), jnp.float32),
            pltpu.VMEM((CAND_CAP,), jnp.float32),
            pltpu.VMEM((CAND_CAP,), jnp.int32),
            pltpu.VMEM((SPW * K + 32,), jnp.int32),
            pltpu.VMEM((SPW * K // 128, 128), jnp.int32),
            pltpu.VMEM((SPW * K, 16), jnp.float32),
            pltpu.SemaphoreType.DMA,
            pltpu.SemaphoreType.DMA,
            pltpu.SemaphoreType.DMA,
        ],
    )
    return f(d2, pos16)


E_REAL = float(N_SEEDS * K)
SBLK = 512
NSB = NSPAD // SBLK


def _moments_body(posj_ref, seeds_ref, w1a_ref, b1a_ref, g1_ref, be1_ref,
                  scale_ref, shift_ref, s1_ref, m2_ref):
    k = pl.program_id(0)
    pj = posj_ref[...]
    sd = seeds_ref[...]
    row = lax.broadcasted_iota(jnp.int32, (NSPAD, 1), 0)
    msk = (row < N_SEEDS).astype(jnp.float32)
    msg = (pj - sd) * msk
    s1 = jnp.sum(msg, axis=0, keepdims=True)
    m2 = lax.dot_general(msg, msg, (((0,), (0,)), ((), ())),
                         preferred_element_type=jnp.float32)

    @pl.when(k == 0)
    def _():
        s1_ref[...] = s1
        m2_ref[...] = m2

    @pl.when(k > 0)
    def _():
        s1_ref[...] = s1_ref[...] + s1
        m2_ref[...] = m2_ref[...] + m2

    @pl.when(k == K - 1)
    def _():
        w = w1a_ref[...]
        s1t = s1_ref[...]
        m2t = m2_ref[...]
        mean_raw = jnp.dot(s1t, w, preferred_element_type=jnp.float32) / E_REAL
        m2w = jnp.dot(m2t, w, preferred_element_type=jnp.float32)
        q = jnp.sum(w * m2w, axis=0, keepdims=True)
        b1 = b1a_ref[...]
        mean = mean_raw + b1
        ex2 = q / E_REAL + 2.0 * b1 * mean_raw + b1 * b1
        var = ex2 - mean * mean
        scale = g1_ref[...] / jnp.sqrt(var + 1e-5)
        scale_ref[...] = scale
        shift_ref[...] = be1_ref[...] - mean * scale


def _bn1_stats(posj, seedsX, W1aP, b1a2, g12, be12):
    return pl.pallas_call(
        _moments_body,
        grid=(K,),
        in_specs=[pl.BlockSpec((NSPAD, 16), lambda k: (k, 0)),
                  pl.BlockSpec((NSPAD, 16), lambda k: (0, 0)),
                  pl.BlockSpec((16, 128), lambda k: (0, 0)),
                  pl.BlockSpec((1, 128), lambda k: (0, 0)),
                  pl.BlockSpec((1, 128), lambda k: (0, 0)),
                  pl.BlockSpec((1, 128), lambda k: (0, 0))],
        out_specs=[pl.BlockSpec((1, 128), lambda k: (0, 0)),
                   pl.BlockSpec((1, 128), lambda k: (0, 0)),
                   pl.BlockSpec((1, 16), lambda k: (0, 0)),
                   pl.BlockSpec((16, 16), lambda k: (0, 0))],
        out_shape=[jax.ShapeDtypeStruct((1, 128), jnp.float32),
                   jax.ShapeDtypeStruct((1, 128), jnp.float32),
                   jax.ShapeDtypeStruct((1, 16), jnp.float32),
                   jax.ShapeDtypeStruct((16, 16), jnp.float32)],
    )(posj, seedsX, W1aP, b1a2, g12, be12)


def _pass1_body(posj_ref, seeds_ref, w1a_ref, b1a_ref, sc1_ref, sh1_ref,
                w1b_ref, b1b_ref, h1_ref, hmax_ref, hsum_ref, a_ref, acc_ref):
    s = pl.program_id(0)
    k = pl.program_id(1)
    pj = posj_ref[...]
    sd = seeds_ref[...]
    msg = pj - sd
    pre = (jnp.dot(msg, w1a_ref[...], preferred_element_type=jnp.float32)
           + b1a_ref[...])
    h = jnp.maximum(pre * sc1_ref[...] + sh1_ref[...], 0.0)
    h1 = (jnp.dot(h, w1b_ref[...], preferred_element_type=jnp.float32)
          + b1b_ref[...])
    row = s * SBLK + lax.broadcasted_iota(jnp.int32, (SBLK, 1), 0)
    h1 = jnp.where(row < N_SEEDS, h1, 0.0)
    h1_ref[...] = h1

    @pl.when(k == 0)
    def _():
        hmax_ref[...] = h1
        hsum_ref[...] = h1

    @pl.when(k > 0)
    def _():
        hmax_ref[...] = jnp.maximum(hmax_ref[...], h1)
        hsum_ref[...] = hsum_ref[...] + h1

    a_blk = lax.dot_general(h1, h1, (((0,), (0,)), ((), ())),
                            preferred_element_type=jnp.float32)

    @pl.when((s == 0) & (k == 0))
    def _():
        acc_ref[...] = a_blk

    @pl.when((s > 0) | (k > 0))
    def _():
        acc_ref[...] = acc_ref[...] + a_blk

    @pl.when((s == NSB - 1) & (k == K - 1))
    def _():
        a_ref[...] = acc_ref[...]


def _pass1(posj, seedsX, W1aP, b1a2, scale1, shift1, W1b, b1b2):
    return pl.pallas_call(
        _pass1_body,
        grid=(NSB, K),
        in_specs=[pl.BlockSpec((SBLK, 16), lambda s, k: (k * NSB + s, 0)),
                  pl.BlockSpec((SBLK, 16), lambda s, k: (s, 0)),
                  pl.BlockSpec((16, 128), lambda s, k: (0, 0)),
                  pl.BlockSpec((1, 128), lambda s, k: (0, 0)),
                  pl.BlockSpec((1, 128), lambda s, k: (0, 0)),
                  pl.BlockSpec((1, 128), lambda s, k: (0, 0)),
                  pl.BlockSpec((128, 256), lambda s, k: (0, 0)),
                  pl.BlockSpec((1, 256), lambda s, k: (0, 0))],
        out_specs=[pl.BlockSpec((SBLK, 256), lambda s, k: (k * NSB + s, 0)),
                   pl.BlockSpec((SBLK, 256), lambda s, k: (s, 0)),
                   pl.BlockSpec((SBLK, 256), lambda s, k: (s, 0)),
                   pl.BlockSpec((256, 256), lambda s, k: (0, 0))],
        out_shape=[jax.ShapeDtypeStruct((K * NSPAD, 256), jnp.float32),
                   jax.ShapeDtypeStruct((NSPAD, 256), jnp.float32),
                   jax.ShapeDtypeStruct((NSPAD, 256), jnp.float32),
                   jax.ShapeDtypeStruct((256, 256), jnp.float32)],
        scratch_shapes=[pltpu.VMEM((256, 256), jnp.float32)],
    )(posj, seedsX, W1aP, b1a2, scale1, shift1, W1b, b1b2)


def _stats2_body(hmax_ref, hsum_ref, a_ref, w2a_ref, b2a_ref, g2_ref, be2_ref,
                 sc2_ref, sh2_ref):
    hmax = hmax_ref[...]
    hsum = hsum_ref[...]
    d = K * lax.dot_general(hmax, hmax, (((0,), (0,)), ((), ())),
                            preferred_element_type=jnp.float32)
    b = lax.dot_general(hmax, hsum, (((0,), (0,)), ((), ())),
                        preferred_element_type=jnp.float32)
    a = a_ref[...]
    top = jnp.concatenate([d, b], axis=1)
    bot = jnp.concatenate([b.T, a], axis=1)
    c = jnp.concatenate([top, bot], axis=0)
    shm = jnp.sum(hmax, axis=0, keepdims=True)
    sh1 = jnp.sum(hsum, axis=0, keepdims=True)
    mean_h = jnp.concatenate([K * shm, sh1], axis=1) / E_REAL
    w = w2a_ref[...]
    mean_raw = jnp.dot(mean_h, w, preferred_element_type=jnp.float32)
    m = jnp.dot(c, w, preferred_element_type=jnp.float32)
    q = jnp.sum(w * m, axis=0, keepdims=True)
    b2 = b2a_ref[...]
    mean2 = mean_raw + b2
    ex2 = q / E_REAL + 2.0 * b2 * mean_raw + b2 * b2
    var = ex2 - mean2 * mean2
    scale = g2_ref[...] / jnp.sqrt(var + 1e-5)
    sc2_ref[...] = scale
    sh2_ref[...] = be2_ref[...] - mean_raw * scale


def _stats2(hmax, hsum, A, W2a, b2a2, g22, be22):
    return pl.pallas_call(
        _stats2_body,
        out_shape=[jax.ShapeDtypeStruct((1, 512), jnp.float32),
                   jax.ShapeDtypeStruct((1, 512), jnp.float32)],
    )(hmax, hsum, A, W2a, b2a2, g22, be22)


def _pass2_body(h1_ref, hmax_ref, sc2_ref, sh2_ref, w2a_ref,
                w2b_ref, b2b_ref, out_ref):
    k = pl.program_id(1)
    h = jnp.concatenate([hmax_ref[...], h1_ref[...]], axis=1)
    pre = jnp.dot(h, w2a_ref[...], preferred_element_type=jnp.float32)
    t = jnp.maximum(pre * sc2_ref[...] + sh2_ref[...], 0.0)
    h2 = (jnp.dot(t, w2b_ref[...], preferred_element_type=jnp.float32)
          + b2b_ref[...])

    @pl.when(k == 0)
    def _():
        out_ref[...] = h2

    @pl.when(k > 0)
    def _():
        out_ref[...] = jnp.maximum(out_ref[...], h2)


def _pass2(h1, hmax, scale2, shift2, W2a, W2b, b2b2):
    return pl.pallas_call(
        _pass2_body,
        grid=(NSB, K),
        in_specs=[pl.BlockSpec((SBLK, 256), lambda s, k: (k * NSB + s, 0)),
                  pl.BlockSpec((SBLK, 256), lambda s, k: (s, 0)),
                  pl.BlockSpec((1, 512), lambda s, k: (0, 0)),
                  pl.BlockSpec((1, 512), lambda s, k: (0, 0)),
                  pl.BlockSpec((512, 512), lambda s, k: (0, 0)),
                  pl.BlockSpec((512, 128), lambda s, k: (0, 0)),
                  pl.BlockSpec((1, 128), lambda s, k: (0, 0))],
        out_specs=pl.BlockSpec((SBLK, 128), lambda s, k: (s, 0)),
        out_shape=jax.ShapeDtypeStruct((NSPAD, 128), jnp.float32),
    )(h1, hmax, scale2, shift2, W2a, W2b, b2b2)


def _mlp_tail(posj, seedsX, W1a, b1a, g1, be1, W1b, b1b, W2a, b2a, g2, be2, W2b, b2b):
    W1aP = jnp.concatenate([W1a, jnp.zeros((13, 128), jnp.float32)], axis=0)
    r2 = lambda v: v.reshape(1, -1)
    scale1, shift1, _, _ = _bn1_stats(posj, seedsX, W1aP, r2(b1a), r2(g1), r2(be1))
    h1, hmax, hsum, A = _pass1(posj, seedsX, W1aP, r2(b1a), scale1, shift1,
                               W1b, r2(b1b))
    scale2, shift2 = _stats2(hmax, hsum, A, W2a, r2(b2a), r2(g2), r2(be2))
    out = _pass2(h1, hmax, scale2, shift2, W2a, W2b, r2(b2b))
    return out[:N_SEEDS]


def kernel(x, pos, batch, W1a, b1a, g1, be1, W1b, b1b, W2a, b2a, g2, be2, W2b, b2b):
    si, sx, sy, sz = _fps_pallas(pos, N_SEEDS)
    sxf = sx.reshape(-1)
    syf = sy.reshape(-1)
    szf = sz.reshape(-1)
    ssq = sxf * sxf + syf * syf + szf * szf
    ones = jnp.ones((NSPAD,), jnp.float32)
    zero = jnp.zeros((NSPAD,), jnp.float32)
    S = jnp.stack([sxf, syf, szf, ssq, ones, zero, zero, zero], axis=1)

    padn = jnp.zeros((NPAD - N_NODES,), jnp.float32)
    px = jnp.concatenate([pos[:, 0], padn])
    py = jnp.concatenate([pos[:, 1], padn])
    pz = jnp.concatenate([pos[:, 2], padn])
    psq = px * px + py * py + pz * pz
    onesp = jnp.ones((NPAD,), jnp.float32)
    zerop = jnp.zeros((NPAD,), jnp.float32)
    P = jnp.stack([-2.0 * px, -2.0 * py, -2.0 * pz, onesp, psq,
                   zerop, zerop, zerop], axis=0)

    seeds = jnp.stack([sxf[:N_SEEDS], syf[:N_SEEDS], szf[:N_SEEDS]], axis=1)
    d2s = (jnp.sum(seeds ** 2, axis=1, keepdims=True)
           + jnp.sum(pos ** 2, axis=1)[None, :]
           - 2.0 * seeds @ pos.T)
    _, nbr = lax.top_k(-d2s, K)
    nbrT = jnp.zeros((K, NSPAD), jnp.int32).at[:, :N_SEEDS].set(nbr.T)
    posj = jnp.zeros((K * NSPAD, 16), jnp.float32).at[:, :3].set(
        pos[nbrT.reshape(-1)])
    seedsX = jnp.zeros((NSPAD, 16), jnp.float32)
    seedsX = seedsX.at[:, 0].set(sxf).at[:, 1].set(syf).at[:, 2].set(szf)
    return _mlp_tail(posj, seedsX, W1a, b1a, g1, be1, W1b, b1b,
                     W2a, b2a, g2, be2, W2b, b2b)
